# baseline (speedup 1.0000x reference)
import jax
import jax.numpy as jnp
from jax import lax
from jax.experimental import pallas as pl
from jax.experimental.pallas import tpu as pltpu
from jax.experimental.pallas import tpu_sc as plsc

NUM_DFAS = 128
NUM_STATES = 64
ALPHABET = 128
BATCH = 2048
SEQ_LEN = 256

NC = 2
NS = 16
L = 16
NW = NC * NS
DPW = NUM_DFAS // NW
TSIZE = NUM_STATES * ALPHABET
GROUPS = 4
NCHAIN = GROUPS * DPW
BC = 256
NCHUNK = BATCH // BC
NQUAD = BC // (GROUPS * L)


def _dfa_body(x8_h, init_h, acc_h, tab_h, out_h, table_v, acc_v, init_v,
              xbuf, out_v):
    wid = lax.axis_index("s") * NC + lax.axis_index("c")
    d0 = wid * DPW

    pltpu.sync_copy(tab_h.at[pl.ds(d0 * TSIZE, DPW * TSIZE)], table_v)
    pltpu.sync_copy(acc_h.at[pl.ds(d0 * NUM_STATES, DPW * NUM_STATES)], acc_v)
    pltpu.sync_copy(init_h, init_v)

    si = [plsc.load_gather(init_v, [jnp.full((L,), d0 + j, jnp.int32)])
          * ALPHABET + j * TSIZE
          for j in range(DPW)]
    si = tuple(si * GROUPS)

    for chunk in range(NCHUNK):
        b0 = chunk * BC
        pltpu.sync_copy(
            x8_h.at[pl.ds(chunk * SEQ_LEN * BC // 4, SEQ_LEN * BC // 4)],
            xbuf.at[pl.ds(0, SEQ_LEN * BC // 4)])

        def bg_body(bg, carry):
            c32 = xbuf[pl.ds(bg * L, L)]
            c0 = tuple(lax.shift_right_logical(c32, 8 * g) & 127
                       for g in range(GROUPS))

            def t_body(t, sc):
                states, c = sc[:NCHAIN], sc[NCHAIN:]
                c32n = xbuf[pl.ds((t + 1) * (BC // 4) + bg * L, L)]
                new = tuple(
                    plsc.load_gather(table_v, [states[k] + c[k // DPW]])
                    for k in range(NCHAIN))
                cn = tuple(lax.shift_right_logical(c32n, 8 * g) & 127
                           for g in range(GROUPS))
                return new + cn

            states = lax.fori_loop(0, SEQ_LEN, t_body, si + c0,
                                   unroll=2)[:NCHAIN]
            for k in range(NCHAIN):
                a = plsc.load_gather(
                    acc_v, [lax.shift_right_logical(states[k], 7)])
                out_v[k % DPW,
                      pl.ds(bg * GROUPS * L + (k // DPW) * L, L)] = a
            return carry

        lax.fori_loop(0, NQUAD, bg_body, 0)
        pltpu.sync_copy(out_v, out_h.at[pl.ds(d0, DPW), pl.ds(b0, BC)])


@jax.jit
def _run(x8, init, acc_i32, tab_scaled):
    mesh = plsc.VectorSubcoreMesh(core_axis_name="c", subcore_axis_name="s",
                                  num_cores=NC, num_subcores=NS)
    f = pl.kernel(
        _dfa_body,
        out_type=jax.ShapeDtypeStruct((NUM_DFAS, BATCH), jnp.int32),
        mesh=mesh,
        scratch_types=[
            pltpu.VMEM((DPW * TSIZE,), jnp.int32),
            pltpu.VMEM((DPW * NUM_STATES,), jnp.int32),
            pltpu.VMEM((NUM_DFAS,), jnp.int32),
            pltpu.VMEM((SEQ_LEN * BC // 4 + BC // 4,), jnp.int32),
            pltpu.VMEM((DPW, BC), jnp.int32),
        ],
        compiler_params=pltpu.CompilerParams(needs_layout_passes=False),
    )
    return f(x8, init, acc_i32, tab_scaled)


def kernel(x, initial_state, accepting_states, transition_function):
    x8 = (x.T.reshape(SEQ_LEN, BATCH // 64, GROUPS, L)
          .transpose(0, 1, 3, 2)
          .reshape(SEQ_LEN, NCHUNK, BC)
          .transpose(1, 0, 2)
          .reshape(NCHUNK * SEQ_LEN * BC // 4, 4)
          .astype(jnp.int8))
    x8 = jax.lax.bitcast_convert_type(x8, jnp.int32)
    acc_i32 = accepting_states.astype(jnp.int32).reshape(-1)
    dlocal = (jnp.arange(NUM_DFAS, dtype=jnp.int32) % DPW)[:, None, None]
    tab_scaled = (transition_function * ALPHABET + dlocal * TSIZE).reshape(-1)
    out = _run(x8, initial_state, acc_i32, tab_scaled)
    return out.astype(jnp.bool_)


# speedup vs baseline: 1.0133x; 1.0133x over previous
import jax
import jax.numpy as jnp
from jax import lax
from jax.experimental import pallas as pl
from jax.experimental.pallas import tpu as pltpu
from jax.experimental.pallas import tpu_sc as plsc

NUM_DFAS = 128
NUM_STATES = 64
ALPHABET = 128
BATCH = 2048
SEQ_LEN = 256

NC = 2
NS = 16
L = 16
NW = NC * NS
DPW = NUM_DFAS // NW
TSIZE = NUM_STATES * ALPHABET
GROUPS = 8
NCHAIN = GROUPS * DPW
BC = 512
NCHUNK = BATCH // BC
NQUAD = BC // (GROUPS * L)


def _dfa_body(x8_h, init_h, acc_h, tab_h, out_h, table_v, acc_v, init_v,
              xbuf, out_v):
    wid = lax.axis_index("s") * NC + lax.axis_index("c")
    d0 = wid * DPW

    pltpu.sync_copy(tab_h.at[pl.ds(d0 * TSIZE, DPW * TSIZE)], table_v)
    pltpu.sync_copy(acc_h.at[pl.ds(d0 * NUM_STATES, DPW * NUM_STATES)], acc_v)
    pltpu.sync_copy(init_h, init_v)

    si = [plsc.load_gather(init_v, [jnp.full((L,), d0 + j, jnp.int32)])
          * ALPHABET + j * TSIZE
          for j in range(DPW)]
    si = tuple(si * GROUPS)

    for chunk in range(NCHUNK):
        b0 = chunk * BC
        pltpu.sync_copy(
            x8_h.at[pl.ds(chunk * SEQ_LEN * BC // 4, SEQ_LEN * BC // 4)],
            xbuf.at[pl.ds(0, SEQ_LEN * BC // 4)])

        def bg_body(bg, carry):
            c32a = xbuf[pl.ds(bg * 2 * L, L)]
            c32b = xbuf[pl.ds(bg * 2 * L + L, L)]
            c0 = tuple(lax.shift_right_logical(w, 8 * g) & 127
                       for w in (c32a, c32b) for g in range(4))

            def t_body(t, sc):
                states, c = sc[:NCHAIN], sc[NCHAIN:]
                c32an = xbuf[pl.ds((t + 1) * (BC // 4) + bg * 2 * L, L)]
                c32bn = xbuf[pl.ds((t + 1) * (BC // 4) + bg * 2 * L + L, L)]
                new = tuple(
                    plsc.load_gather(table_v, [states[k] + c[k // DPW]])
                    for k in range(NCHAIN))
                cn = tuple(lax.shift_right_logical(w, 8 * g) & 127
                           for w in (c32an, c32bn) for g in range(4))
                return new + cn

            states = lax.fori_loop(0, SEQ_LEN, t_body, si + c0,
                                   unroll=1)[:NCHAIN]
            for k in range(NCHAIN):
                a = plsc.load_gather(
                    acc_v, [lax.shift_right_logical(states[k], 7)])
                out_v[k % DPW,
                      pl.ds(bg * GROUPS * L + (k // DPW) * L, L)] = a
            return carry

        lax.fori_loop(0, NQUAD, bg_body, 0)
        pltpu.sync_copy(out_v, out_h.at[pl.ds(d0, DPW), pl.ds(b0, BC)])


@jax.jit
def _run(x8, init, acc_i32, tab_scaled):
    mesh = plsc.VectorSubcoreMesh(core_axis_name="c", subcore_axis_name="s",
                                  num_cores=NC, num_subcores=NS)
    f = pl.kernel(
        _dfa_body,
        out_type=jax.ShapeDtypeStruct((NUM_DFAS, BATCH), jnp.int32),
        mesh=mesh,
        scratch_types=[
            pltpu.VMEM((DPW * TSIZE,), jnp.int32),
            pltpu.VMEM((DPW * NUM_STATES,), jnp.int32),
            pltpu.VMEM((NUM_DFAS,), jnp.int32),
            pltpu.VMEM((SEQ_LEN * BC // 4 + BC // 4,), jnp.int32),
            pltpu.VMEM((DPW, BC), jnp.int32),
        ],
        compiler_params=pltpu.CompilerParams(needs_layout_passes=False),
    )
    return f(x8, init, acc_i32, tab_scaled)


def kernel(x, initial_state, accepting_states, transition_function):
    x8 = (x.T.reshape(SEQ_LEN, BATCH // 64, 4, L)
          .transpose(0, 1, 3, 2)
          .reshape(SEQ_LEN, NCHUNK, BC)
          .transpose(1, 0, 2)
          .reshape(NCHUNK * SEQ_LEN * BC // 4, 4)
          .astype(jnp.int8))
    x8 = jax.lax.bitcast_convert_type(x8, jnp.int32)
    acc_i32 = accepting_states.astype(jnp.int32).reshape(-1)
    dlocal = (jnp.arange(NUM_DFAS, dtype=jnp.int32) % DPW)[:, None, None]
    tab_scaled = (transition_function * ALPHABET + dlocal * TSIZE).reshape(-1)
    out = _run(x8, initial_state, acc_i32, tab_scaled)
    return out.astype(jnp.bool_)


# speedup vs baseline: 1.2476x; 1.2312x over previous
import jax
import jax.numpy as jnp
from jax import lax
from jax.experimental import pallas as pl
from jax.experimental.pallas import tpu as pltpu
from jax.experimental.pallas import tpu_sc as plsc

NUM_DFAS = 128
NUM_STATES = 64
ALPHABET = 128
BATCH = 2048
SEQ_LEN = 256

NC = 2
NS = 16
L = 16
NW = NC * NS
DPW = NUM_DFAS // NW
TSIZE = NUM_STATES * ALPHABET
GROUPS = 4
NCHAIN = GROUPS * DPW
BC = 256
NCHUNK = BATCH // BC
NQUAD = BC // (GROUPS * L)


def _dfa_body(xT_h, init_h, acc_h, tab_h, out_h, table_v, acc_v, init_v,
              xbuf, out_v):
    wid = lax.axis_index("s") * NC + lax.axis_index("c")
    d0 = wid * DPW

    pltpu.sync_copy(tab_h.at[pl.ds(d0 * TSIZE, DPW * TSIZE)], table_v)
    pltpu.sync_copy(acc_h.at[pl.ds(d0 * NUM_STATES, DPW * NUM_STATES)], acc_v)
    pltpu.sync_copy(init_h, init_v)

    si = [plsc.load_gather(init_v, [jnp.full((L,), d0 + j, jnp.int32)])
          * ALPHABET + j * TSIZE
          for j in range(DPW)]
    si = tuple(si * GROUPS)

    for chunk in range(NCHUNK):
        b0 = chunk * BC
        pltpu.sync_copy(xT_h.at[:, pl.ds(b0, BC)], xbuf)

        def bg_body(bg, carry):
            def t_body(t, states):
                c = [xbuf[t, pl.ds(bg * GROUPS * L + g * L, L)]
                     for g in range(GROUPS)]
                return tuple(
                    plsc.load_gather(table_v, [states[k] + c[k // DPW]])
                    for k in range(NCHAIN))

            states = lax.fori_loop(0, SEQ_LEN, t_body, si, unroll=2)
            for k in range(NCHAIN):
                a = plsc.load_gather(
                    acc_v, [lax.shift_right_logical(states[k], 7)])
                out_v[k % DPW,
                      pl.ds(bg * GROUPS * L + (k // DPW) * L, L)] = a
            return carry

        lax.fori_loop(0, NQUAD, bg_body, 0)
        pltpu.sync_copy(out_v, out_h.at[pl.ds(d0, DPW), pl.ds(b0, BC)])


@jax.jit
def _run(xT, init, acc_i32, tab_scaled):
    mesh = plsc.VectorSubcoreMesh(core_axis_name="c", subcore_axis_name="s",
                                  num_cores=NC, num_subcores=NS)
    f = pl.kernel(
        _dfa_body,
        out_type=jax.ShapeDtypeStruct((NUM_DFAS, BATCH), jnp.int32),
        mesh=mesh,
        scratch_types=[
            pltpu.VMEM((DPW * TSIZE,), jnp.int32),
            pltpu.VMEM((DPW * NUM_STATES,), jnp.int32),
            pltpu.VMEM((NUM_DFAS,), jnp.int32),
            pltpu.VMEM((SEQ_LEN, BC), jnp.int32),
            pltpu.VMEM((DPW, BC), jnp.int32),
        ],
        compiler_params=pltpu.CompilerParams(needs_layout_passes=False),
    )
    return f(xT, init, acc_i32, tab_scaled)


def kernel(x, initial_state, accepting_states, transition_function):
    xT = x.T.reshape(SEQ_LEN, BATCH)
    acc_i32 = accepting_states.astype(jnp.int32).reshape(-1)
    dlocal = (jnp.arange(NUM_DFAS, dtype=jnp.int32) % DPW)[:, None, None]
    tab_scaled = (transition_function * ALPHABET + dlocal * TSIZE).reshape(-1)
    out = _run(xT, initial_state, acc_i32, tab_scaled)
    return out.astype(jnp.bool_)


# speedup vs baseline: 1.4265x; 1.1435x over previous
import jax
import jax.numpy as jnp
from jax import lax
from jax.experimental import pallas as pl
from jax.experimental.pallas import tpu as pltpu
from jax.experimental.pallas import tpu_sc as plsc

NUM_DFAS = 128
NUM_STATES = 64
ALPHABET = 128
BATCH = 2048
SEQ_LEN = 256

NC = 2
NS = 16
L = 16
NW = NC * NS
DPW = NUM_DFAS // NW
TSIZE = NUM_STATES * ALPHABET
GROUPS = 4
NCHAIN = GROUPS * DPW
BC = 128
NCHUNK = BATCH // BC
NQUAD = BC // (GROUPS * L)


def _dfa_body(xT_h, init_h, acc_h, tab_h, out_h, table_v, acc_v, init_v,
              xbuf0, xbuf1, out_v, xs0, xs1, os0, os1):
    wid = lax.axis_index("s") * NC + lax.axis_index("c")
    d0 = wid * DPW

    xbufs = [xbuf0, xbuf1]
    xsems = [xs0, xs1]
    osems = [os0, os1]

    xd = {0: pltpu.async_copy(xT_h.at[:, pl.ds(0, BC)], xbuf0, xs0)}
    pltpu.sync_copy(tab_h.at[pl.ds(d0 * TSIZE, DPW * TSIZE)], table_v)
    pltpu.sync_copy(acc_h.at[pl.ds(d0 * NUM_STATES, DPW * NUM_STATES)], acc_v)
    pltpu.sync_copy(init_h, init_v)

    si = [plsc.load_gather(init_v, [jnp.full((L,), d0 + j, jnp.int32)])
          * ALPHABET + j * TSIZE
          for j in range(DPW)]
    si = tuple(si * GROUPS)

    od = {}
    for chunk in range(NCHUNK):
        cur = chunk % 2
        b0 = chunk * BC
        if chunk + 1 < NCHUNK:
            xd[chunk + 1] = pltpu.async_copy(
                xT_h.at[:, pl.ds(b0 + BC, BC)], xbufs[1 - cur],
                xsems[1 - cur])
        xd[chunk].wait()
        xbuf = xbufs[cur]

        def bg_body(bg, carry):
            def t_body(t, states):
                c = [xbuf[t, pl.ds(bg * GROUPS * L + g * L, L)]
                     for g in range(GROUPS)]
                return tuple(
                    plsc.load_gather(table_v, [states[k] + c[k // DPW]])
                    for k in range(NCHAIN))

            states = lax.fori_loop(0, SEQ_LEN, t_body, si, unroll=2)
            for k in range(NCHAIN):
                a = plsc.load_gather(
                    acc_v, [lax.shift_right_logical(states[k], 7)])
                out_v[cur, k % DPW,
                      pl.ds(bg * GROUPS * L + (k // DPW) * L, L)] = a
            return carry

        lax.fori_loop(0, NQUAD, bg_body, 0)
        if chunk >= 2:
            od[chunk - 2].wait()
        od[chunk] = pltpu.async_copy(
            out_v.at[cur], out_h.at[pl.ds(d0, DPW), pl.ds(b0, BC)],
            osems[cur])
    od[NCHUNK - 2].wait()
    od[NCHUNK - 1].wait()


@jax.jit
def _run(xT, init, acc_i32, tab_scaled):
    mesh = plsc.VectorSubcoreMesh(core_axis_name="c", subcore_axis_name="s",
                                  num_cores=NC, num_subcores=NS)
    f = pl.kernel(
        _dfa_body,
        out_type=jax.ShapeDtypeStruct((NUM_DFAS, BATCH), jnp.int32),
        mesh=mesh,
        scratch_types=[
            pltpu.VMEM((DPW * TSIZE,), jnp.int32),
            pltpu.VMEM((DPW * NUM_STATES,), jnp.int32),
            pltpu.VMEM((NUM_DFAS,), jnp.int32),
            pltpu.VMEM((SEQ_LEN, BC), jnp.int32),
            pltpu.VMEM((SEQ_LEN, BC), jnp.int32),
            pltpu.VMEM((2, DPW, BC), jnp.int32),
            pltpu.SemaphoreType.DMA,
            pltpu.SemaphoreType.DMA,
            pltpu.SemaphoreType.DMA,
            pltpu.SemaphoreType.DMA,
        ],
        compiler_params=pltpu.CompilerParams(needs_layout_passes=False),
    )
    return f(xT, init, acc_i32, tab_scaled)


def kernel(x, initial_state, accepting_states, transition_function):
    xT = x.T.reshape(SEQ_LEN, BATCH)
    acc_i32 = accepting_states.astype(jnp.int32).reshape(-1)
    dlocal = (jnp.arange(NUM_DFAS, dtype=jnp.int32) % DPW)[:, None, None]
    tab_scaled = (transition_function * ALPHABET + dlocal * TSIZE).reshape(-1)
    out = _run(xT, initial_state, acc_i32, tab_scaled)
    return out.astype(jnp.bool_)


# speedup vs baseline: 1.4897x; 1.0443x over previous
import jax
import jax.numpy as jnp
from jax import lax
from jax.experimental import pallas as pl
from jax.experimental.pallas import tpu as pltpu
from jax.experimental.pallas import tpu_sc as plsc

NUM_DFAS = 128
NUM_STATES = 64
ALPHABET = 128
BATCH = 2048
SEQ_LEN = 256

NC = 2
NS = 16
L = 16
NW = NC * NS
DPW = NUM_DFAS // NW
TSIZE = NUM_STATES * ALPHABET
GROUPS = 4
NCHAIN = GROUPS * DPW
BC = 128
NCHUNK = BATCH // BC
NQUAD = BC // (GROUPS * L)


def _dfa_body(xT_h, init_h, acc_h, tab_h, out_h, table_v, acc_v, init_v,
              xbuf0, xbuf1, out_v, xs0, xs1, os0, os1):
    wid = lax.axis_index("s") * NC + lax.axis_index("c")
    d0 = wid * DPW

    xbufs = [xbuf0, xbuf1]
    xsems = [xs0, xs1]
    osems = [os0, os1]

    xd = {0: pltpu.async_copy(xT_h.at[:, pl.ds(0, BC)], xbuf0, xs0)}
    pltpu.sync_copy(tab_h.at[pl.ds(d0 * TSIZE, DPW * TSIZE)], table_v)
    pltpu.sync_copy(acc_h.at[pl.ds(d0 * NUM_STATES, DPW * NUM_STATES)], acc_v)
    pltpu.sync_copy(init_h, init_v)

    si = [plsc.load_gather(init_v, [jnp.full((L,), d0 + j, jnp.int32)])
          * ALPHABET + j * TSIZE
          for j in range(DPW)]
    si = tuple(si * GROUPS)

    od = {}
    for chunk in range(NCHUNK):
        cur = chunk % 2
        b0 = chunk * BC
        if chunk + 1 < NCHUNK:
            xd[chunk + 1] = pltpu.async_copy(
                xT_h.at[:, pl.ds(b0 + BC, BC)], xbufs[1 - cur],
                xsems[1 - cur])
        xd[chunk].wait()
        xbuf = xbufs[cur]

        def bg_body(bg, carry):
            def t_body(t, states):
                c = [xbuf[t, pl.ds(bg * GROUPS * L + g * L, L)]
                     for g in range(GROUPS)]
                return tuple(
                    plsc.load_gather(table_v, [states[k] + c[k // DPW]])
                    for k in range(NCHAIN))

            states = lax.fori_loop(0, SEQ_LEN, t_body, si, unroll=4)
            for k in range(NCHAIN):
                a = plsc.load_gather(
                    acc_v, [lax.shift_right_logical(states[k], 7)])
                out_v[cur, k % DPW,
                      pl.ds(bg * GROUPS * L + (k // DPW) * L, L)] = a
            return carry

        lax.fori_loop(0, NQUAD, bg_body, 0)
        if chunk >= 2:
            od[chunk - 2].wait()
        od[chunk] = pltpu.async_copy(
            out_v.at[cur], out_h.at[pl.ds(d0, DPW), pl.ds(b0, BC)],
            osems[cur])
    od[NCHUNK - 2].wait()
    od[NCHUNK - 1].wait()


@jax.jit
def _run(xT, init, acc_i32, tab_scaled):
    mesh = plsc.VectorSubcoreMesh(core_axis_name="c", subcore_axis_name="s",
                                  num_cores=NC, num_subcores=NS)
    f = pl.kernel(
        _dfa_body,
        out_type=jax.ShapeDtypeStruct((NUM_DFAS, BATCH), jnp.int32),
        mesh=mesh,
        scratch_types=[
            pltpu.VMEM((DPW * TSIZE,), jnp.int32),
            pltpu.VMEM((DPW * NUM_STATES,), jnp.int32),
            pltpu.VMEM((NUM_DFAS,), jnp.int32),
            pltpu.VMEM((SEQ_LEN, BC), jnp.int32),
            pltpu.VMEM((SEQ_LEN, BC), jnp.int32),
            pltpu.VMEM((2, DPW, BC), jnp.int32),
            pltpu.SemaphoreType.DMA,
            pltpu.SemaphoreType.DMA,
            pltpu.SemaphoreType.DMA,
            pltpu.SemaphoreType.DMA,
        ],
        compiler_params=pltpu.CompilerParams(needs_layout_passes=False),
    )
    return f(xT, init, acc_i32, tab_scaled)


def kernel(x, initial_state, accepting_states, transition_function):
    xT = x.T.reshape(SEQ_LEN, BATCH)
    acc_i32 = accepting_states.astype(jnp.int32).reshape(-1)
    dlocal = (jnp.arange(NUM_DFAS, dtype=jnp.int32) % DPW)[:, None, None]
    tab_scaled = (transition_function * ALPHABET + dlocal * TSIZE).reshape(-1)
    out = _run(xT, initial_state, acc_i32, tab_scaled)
    return out.astype(jnp.bool_)
